# baseline (speedup 1.0000x reference)
import jax
import jax.numpy as jnp
from jax import lax
from jax.experimental import pallas as pl
from jax.experimental.pallas import tpu as pltpu
from jax.experimental.pallas import tpu_sc as plsc

N = 10000
D = 128
E = 320000

NC = 2
NS = 16
NW = NC * NS
L = 16
CHUNK = 128
CPW = 79
GC = 64
EPW = CPW * CHUNK
EPAD = NW * EPW
NP = 10112
PAD_ROWS = NP - N
RPT = NP // NS
DEGN = 10240
DRPT = DEGN // NS
MAGIC = 0x5F3759DF


def _rsqrt16(x):
    i = plsc.bitcast(x, jnp.int32)
    i = MAGIC - jnp.right_shift(i, 1)
    h = plsc.bitcast(i, jnp.float32)
    for _ in range(3):
        h = h * (1.5 - 0.5 * x * h * h)
    return h


def _sc_body(xw_hbm, src_hbm, dst_hbm, dst64_hbm, out_hbm, y_hbm,
             sidx, didx64, rows_a, rows_b, degv, disv, ones_v,
             sga, sgb, ssa, ssb, deg_sh, acc_sh):
    c = lax.axis_index("c")
    s = lax.axis_index("s")
    wid = s * NC + c
    abase = s * RPT
    dbase = s * DRPT

    pltpu.sync_copy(dst64_hbm.at[wid], didx64)
    pltpu.sync_copy(dst_hbm.at[s, 1 - c], sidx)

    for k in range(DRPT // L):
        degv[pl.ds(k * L, L)] = jnp.zeros((L,), jnp.float32)
    for k in range(CHUNK // L):
        ones_v[pl.ds(k * L, L)] = jnp.ones((L,), jnp.float32)
    pltpu.sync_copy(degv, deg_sh.at[pl.ds(dbase, DRPT)])
    plsc.subcore_barrier()

    def dbody(j, carry):
        pltpu.async_copy(ones_v, deg_sh.at[sidx.at[j]], sga, add=True)

        @pl.when(j >= 8)
        def _():
            pltpu.make_async_copy(ones_v, deg_sh.at[pl.ds(0, CHUNK)],
                                  sga).wait()

        return carry

    lax.fori_loop(0, CPW, dbody, 0)

    def ddrain(j, carry):
        pltpu.make_async_copy(ones_v, deg_sh.at[pl.ds(0, CHUNK)], sga).wait()
        return carry

    lax.fori_loop(0, 8, ddrain, 0)

    def dbody2(j, carry):
        pltpu.async_copy(ones_v.at[pl.ds(0, GC)], deg_sh.at[didx64.at[j]],
                         sga, add=True)

        @pl.when(j >= 8)
        def _():
            pltpu.make_async_copy(ones_v.at[pl.ds(0, GC)],
                                  deg_sh.at[pl.ds(0, GC)], sga).wait()

        return carry

    lax.fori_loop(0, 2 * CPW, dbody2, 0)

    def ddrain2(j, carry):
        pltpu.make_async_copy(ones_v.at[pl.ds(0, GC)],
                              deg_sh.at[pl.ds(0, GC)], sga).wait()
        return carry

    lax.fori_loop(0, 8, ddrain2, 0)
    plsc.subcore_barrier()

    pltpu.sync_copy(src_hbm.at[wid], sidx)

    pltpu.sync_copy(deg_sh.at[pl.ds(dbase, DRPT)], degv)

    def nbody(k, carry):
        disv[pl.ds(k * L, L)] = _rsqrt16(degv[pl.ds(k * L, L)] + 1.0)
        return carry

    lax.fori_loop(0, DRPT // L, nbody, 0)

    def scale_chunk(r0, nrows):
        pltpu.sync_copy(xw_hbm.at[pl.ds(dbase + r0, nrows)],
                        rows_a.at[pl.ds(0, nrows)])

        def gbody(g, carry):
            d16 = disv[pl.ds(r0 + g * L, L)]
            for i in range(L):
                bi = jnp.broadcast_to(d16[i], (L,))
                row = g * L + i
                for k2 in range(D // L):
                    rows_a[row, pl.ds(k2 * L, L)] = (
                        rows_a[row, pl.ds(k2 * L, L)] * bi)
            return carry

        lax.fori_loop(0, nrows // L, gbody, 0)
        pltpu.sync_copy(rows_a.at[pl.ds(0, nrows)],
                        y_hbm.at[c, pl.ds(dbase + r0, nrows)])

        @pl.when(c == 0)
        def _():
            pltpu.sync_copy(rows_a.at[pl.ds(0, nrows)],
                            acc_sh.at[pl.ds(dbase + r0, nrows)])

    @pl.when(s < NS - 1)
    def _():
        def cbody(j, carry):
            scale_chunk(j * GC, GC)
            return carry

        lax.fori_loop(0, DRPT // GC, cbody, 0)

    @pl.when(s == NS - 1)
    def _():
        def cbody(j, carry):
            scale_chunk(j * GC, GC)
            return carry

        lax.fori_loop(0, 6, cbody, 0)
        scale_chunk(384, L)

    @pl.when(c == 1)
    def _():
        def zrow(i, carry):
            for k in range(D // L):
                rows_a[i, pl.ds(k * L, L)] = jnp.zeros((L,), jnp.float32)
            return carry

        lax.fori_loop(0, GC, zrow, 0)
        nz = jnp.where(s < NS - 1, DRPT // GC, 8)

        def zbody(j, carry):
            pltpu.sync_copy(rows_a, acc_sh.at[pl.ds(dbase + j * GC, GC)])
            return carry

        lax.fori_loop(0, nz, zbody, 0)

    plsc.subcore_barrier()

    yt = y_hbm.at[c]
    pltpu.async_copy(yt.at[sidx.at[0, pl.ds(0, GC)]], rows_a, sga)

    def mbody(jj, carry):
        @pl.when(jj > 0)
        def _():
            pltpu.make_async_copy(rows_b, acc_sh.at[pl.ds(0, GC)],
                                  ssb).wait()

        pltpu.async_copy(yt.at[sidx.at[jj, pl.ds(GC, GC)]], rows_b, sgb)
        pltpu.make_async_copy(y_hbm.at[c, pl.ds(0, GC)], rows_a, sga).wait()
        pltpu.async_copy(rows_a, acc_sh.at[didx64.at[2 * jj]], ssa, add=True)
        pltpu.make_async_copy(y_hbm.at[c, pl.ds(0, GC)], rows_b, sgb).wait()
        pltpu.async_copy(rows_b, acc_sh.at[didx64.at[2 * jj + 1]], ssb,
                         add=True)

        @pl.when(jj < CPW - 1)
        def _():
            pltpu.make_async_copy(rows_a, acc_sh.at[pl.ds(0, GC)],
                                  ssa).wait()
            pltpu.async_copy(yt.at[sidx.at[jj + 1, pl.ds(0, GC)]], rows_a,
                             sga)

        return carry

    lax.fori_loop(0, CPW, mbody, 0)
    pltpu.make_async_copy(rows_a, acc_sh.at[pl.ds(0, GC)], ssa).wait()
    pltpu.make_async_copy(rows_b, acc_sh.at[pl.ds(0, GC)], ssb).wait()
    plsc.subcore_barrier()

    nw = jnp.where(s < NS - 1, DRPT // GC, 8)

    def wbody(j, carry):
        pltpu.sync_copy(acc_sh.at[pl.ds(dbase + j * GC, GC)], rows_a)

        def wg(g, carry2):
            d16 = disv[pl.ds(j * GC + g * L, L)]
            for i in range(L):
                bi = jnp.broadcast_to(d16[i], (L,))
                row = g * L + i
                for k2 in range(D // L):
                    rows_a[row, pl.ds(k2 * L, L)] = (
                        rows_a[row, pl.ds(k2 * L, L)] * bi)
            return carry2

        lax.fori_loop(0, GC // L, wg, 0)
        pltpu.sync_copy(rows_a, out_hbm.at[c, pl.ds(dbase + j * GC, GC)])
        return carry

    lax.fori_loop(0, nw, wbody, 0)


def _mm_body(x_ref, w_ref, y_ref):
    y_ref[...] = jnp.dot(x_ref[...], w_ref[...],
                         preferred_element_type=jnp.float32)


def _fin_body(a_ref, b_ref, o_ref):
    acc = a_ref[0, :, :] + a_ref[1, :, :]
    o_ref[...] = jnp.maximum(acc + b_ref[...], 0.0)


def kernel(x, edge_index, W, b):
    src = edge_index[0].astype(jnp.int32)
    dst = edge_index[1].astype(jnp.int32)
    pad = EPAD - E
    pidx = jnp.arange(pad, dtype=jnp.int32)
    src3 = jnp.concatenate([src, pidx % N]).reshape(NW, CPW, CHUNK)
    dstp = jnp.concatenate([dst, N + pidx % PAD_ROWS])
    dst4 = dstp.reshape(NS, NC, CPW, CHUNK)
    dst64 = dstp.reshape(NW, 2 * CPW, GC)

    xw = pl.pallas_call(
        _mm_body,
        grid=(10,),
        in_specs=[
            pl.BlockSpec((1000, D), lambda i: (i, 0)),
            pl.BlockSpec((D, D), lambda i: (0, 0)),
        ],
        out_specs=pl.BlockSpec((1000, D), lambda i: (i, 0)),
        out_shape=jax.ShapeDtypeStruct((N, D), jnp.float32),
    )(x, W)

    parts, _y = pl.kernel(
        _sc_body,
        out_type=(
            jax.ShapeDtypeStruct((NC, NP, D), jnp.float32),
            jax.ShapeDtypeStruct((NC, N, D), jnp.float32),
        ),
        mesh=plsc.VectorSubcoreMesh(core_axis_name="c", subcore_axis_name="s"),
        scratch_types=[
            pltpu.VMEM((CPW, CHUNK), jnp.int32),
            pltpu.VMEM((2 * CPW, GC), jnp.int32),
            pltpu.VMEM((GC, D), jnp.float32),
            pltpu.VMEM((GC, D), jnp.float32),
            pltpu.VMEM((DRPT,), jnp.float32),
            pltpu.VMEM((DRPT,), jnp.float32),
            pltpu.VMEM((CHUNK,), jnp.float32),
            pltpu.SemaphoreType.DMA,
            pltpu.SemaphoreType.DMA,
            pltpu.SemaphoreType.DMA,
            pltpu.SemaphoreType.DMA,
            pltpu.VMEM_SHARED((DEGN,), jnp.float32),
            pltpu.VMEM_SHARED((NP, D), jnp.float32),
        ],
        compiler_params=pltpu.CompilerParams(needs_layout_passes=False),
    )(xw, src3, dst4, dst64)

    out = pl.pallas_call(
        _fin_body,
        grid=(10,),
        in_specs=[
            pl.BlockSpec((NC, 1000, D), lambda i: (0, i, 0)),
            pl.BlockSpec((1, D), lambda i: (0, 0)),
        ],
        out_specs=pl.BlockSpec((1000, D), lambda i: (i, 0)),
        out_shape=jax.ShapeDtypeStruct((N, D), jnp.float32),
    )(parts, b.reshape(1, D))
    return out


# speedup vs baseline: 1.1327x; 1.1327x over previous
import jax
import jax.numpy as jnp
from jax import lax
from jax.experimental import pallas as pl
from jax.experimental.pallas import tpu as pltpu
from jax.experimental.pallas import tpu_sc as plsc

N = 10000
D = 128
E = 320000

NC = 2
NS = 16
NW = NC * NS
L = 16
CHUNK = 128
CPW = 79
GC = 64
EPW = CPW * CHUNK
EPAD = NW * EPW
NP = 10112
PAD_ROWS = NP - N
RPT = NP // NS
DEGN = 10240
DRPT = DEGN // NS
MAGIC = 0x5F3759DF


def _rsqrt16(x):
    i = plsc.bitcast(x, jnp.int32)
    i = MAGIC - jnp.right_shift(i, 1)
    h = plsc.bitcast(i, jnp.float32)
    for _ in range(3):
        h = h * (1.5 - 0.5 * x * h * h)
    return h


def _sc_body(xw_hbm, src_hbm, dst_hbm, dst64_hbm, out_hbm, y_hbm,
             sidx, didx64, rows_a, rows_b, degv, disv, ones_v,
             sga, sgb, deg_sh, acc_sh):
    c = lax.axis_index("c")
    s = lax.axis_index("s")
    wid = s * NC + c
    abase = s * RPT
    dbase = s * DRPT

    pltpu.sync_copy(dst64_hbm.at[wid], didx64)
    pltpu.sync_copy(dst_hbm.at[s, 1 - c], sidx)

    for k in range(DRPT // L):
        degv[pl.ds(k * L, L)] = jnp.zeros((L,), jnp.float32)
    for k in range(CHUNK // L):
        ones_v[pl.ds(k * L, L)] = jnp.ones((L,), jnp.float32)
    pltpu.sync_copy(degv, deg_sh.at[pl.ds(dbase, DRPT)])
    plsc.subcore_barrier()

    def dbody(j, carry):
        pltpu.sync_copy(ones_v, deg_sh.at[sidx.at[j]], add=True)
        return carry

    lax.fori_loop(0, CPW, dbody, 0)

    def dbody2(j, carry):
        pltpu.sync_copy(ones_v.at[pl.ds(0, GC)], deg_sh.at[didx64.at[j]],
                        add=True)
        return carry

    lax.fori_loop(0, 2 * CPW, dbody2, 0)
    plsc.subcore_barrier()

    pltpu.sync_copy(src_hbm.at[wid], sidx)

    pltpu.sync_copy(deg_sh.at[pl.ds(dbase, DRPT)], degv)

    def nbody(k, carry):
        disv[pl.ds(k * L, L)] = _rsqrt16(degv[pl.ds(k * L, L)] + 1.0)
        return carry

    lax.fori_loop(0, DRPT // L, nbody, 0)

    def scale_buf(buf, r0, nrows):
        def gbody(g, carry):
            d16 = disv[pl.ds(r0 + g * L, L)]
            for i in range(L):
                bi = jnp.broadcast_to(d16[i], (L,))
                row = g * L + i
                for k2 in range(D // L):
                    buf[row, pl.ds(k2 * L, L)] = buf[row, pl.ds(k2 * L, L)] * bi
            return carry

        lax.fori_loop(0, nrows // L, gbody, 0)
        pltpu.sync_copy(buf.at[pl.ds(0, nrows)],
                        y_hbm.at[c, pl.ds(dbase + r0, nrows)])

        @pl.when(c == 0)
        def _():
            pltpu.sync_copy(buf.at[pl.ds(0, nrows)],
                            acc_sh.at[pl.ds(dbase + r0, nrows)])

    def scale_region(npairs):
        pltpu.async_copy(xw_hbm.at[pl.ds(dbase, GC)], rows_a, sga)

        def pbody(jj, carry):
            pltpu.async_copy(xw_hbm.at[pl.ds(dbase + (2 * jj + 1) * GC, GC)],
                             rows_b, sgb)
            pltpu.make_async_copy(xw_hbm.at[pl.ds(0, GC)], rows_a, sga).wait()
            scale_buf(rows_a, 2 * jj * GC, GC)

            @pl.when(jj < npairs - 1)
            def _():
                pltpu.async_copy(
                    xw_hbm.at[pl.ds(dbase + (2 * jj + 2) * GC, GC)], rows_a,
                    sga)

            pltpu.make_async_copy(xw_hbm.at[pl.ds(0, GC)], rows_b, sgb).wait()
            scale_buf(rows_b, (2 * jj + 1) * GC, GC)
            return carry

        lax.fori_loop(0, npairs, pbody, 0)

    @pl.when(s < NS - 1)
    def _():
        scale_region(DRPT // GC // 2)

    @pl.when(s == NS - 1)
    def _():
        scale_region(3)
        pltpu.sync_copy(xw_hbm.at[pl.ds(dbase + 384, L)],
                        rows_a.at[pl.ds(0, L)])
        scale_buf(rows_a, 384, L)

    @pl.when(c == 1)
    def _():
        def zrow(i, carry):
            for k in range(D // L):
                rows_a[i, pl.ds(k * L, L)] = jnp.zeros((L,), jnp.float32)
            return carry

        lax.fori_loop(0, GC, zrow, 0)
        nz = jnp.where(s < NS - 1, DRPT // GC, 8)

        def zbody(j, carry):
            pltpu.sync_copy(rows_a, acc_sh.at[pl.ds(dbase + j * GC, GC)])
            return carry

        lax.fori_loop(0, nz, zbody, 0)

    plsc.subcore_barrier()

    yt = y_hbm.at[c]
    pltpu.async_copy(yt.at[sidx.at[0, pl.ds(0, GC)]], rows_a, sga)

    def mbody(jj, carry):
        pltpu.async_copy(yt.at[sidx.at[jj, pl.ds(GC, GC)]], rows_b, sgb)
        pltpu.make_async_copy(y_hbm.at[c, pl.ds(0, GC)], rows_a, sga).wait()
        pltpu.sync_copy(rows_a, acc_sh.at[didx64.at[2 * jj]], add=True)

        @pl.when(jj < CPW - 1)
        def _():
            pltpu.async_copy(yt.at[sidx.at[jj + 1, pl.ds(0, GC)]], rows_a,
                             sga)

        pltpu.make_async_copy(y_hbm.at[c, pl.ds(0, GC)], rows_b, sgb).wait()
        pltpu.sync_copy(rows_b, acc_sh.at[didx64.at[2 * jj + 1]], add=True)
        return carry

    lax.fori_loop(0, CPW, mbody, 0)
    plsc.subcore_barrier()

    def wb_buf(buf, j):
        def wg(g, carry2):
            d16 = disv[pl.ds(j * GC + g * L, L)]
            for i in range(L):
                bi = jnp.broadcast_to(d16[i], (L,))
                row = g * L + i
                for k2 in range(D // L):
                    buf[row, pl.ds(k2 * L, L)] = buf[row, pl.ds(k2 * L, L)] * bi
            return carry2

        lax.fori_loop(0, GC // L, wg, 0)
        pltpu.sync_copy(buf, out_hbm.at[c, pl.ds(dbase + j * GC, GC)])

    nwp = jnp.where(s < NS - 1, DRPT // GC // 2, 4)
    pltpu.async_copy(acc_sh.at[pl.ds(dbase, GC)], rows_a, sga)

    def wpair(jj, carry):
        pltpu.async_copy(acc_sh.at[pl.ds(dbase + (2 * jj + 1) * GC, GC)],
                         rows_b, sgb)
        pltpu.make_async_copy(acc_sh.at[pl.ds(0, GC)], rows_a, sga).wait()
        wb_buf(rows_a, 2 * jj)

        @pl.when(jj < nwp - 1)
        def _():
            pltpu.async_copy(acc_sh.at[pl.ds(dbase + (2 * jj + 2) * GC, GC)],
                             rows_a, sga)

        pltpu.make_async_copy(acc_sh.at[pl.ds(0, GC)], rows_b, sgb).wait()
        wb_buf(rows_b, 2 * jj + 1)
        return carry

    lax.fori_loop(0, nwp, wpair, 0)


def _mm_body(x_ref, w_ref, y_ref):
    y_ref[...] = jnp.dot(x_ref[...], w_ref[...],
                         preferred_element_type=jnp.float32)


def _fin_body(a_ref, b_ref, o_ref):
    acc = a_ref[0, :, :] + a_ref[1, :, :]
    o_ref[...] = jnp.maximum(acc + b_ref[...], 0.0)


def kernel(x, edge_index, W, b):
    src = edge_index[0].astype(jnp.int32)
    dst = edge_index[1].astype(jnp.int32)
    pad = EPAD - E
    pidx = jnp.arange(pad, dtype=jnp.int32)
    src3 = jnp.concatenate([src, pidx % N]).reshape(NW, CPW, CHUNK)
    dstp = jnp.concatenate([dst, N + pidx % PAD_ROWS])
    dst4 = dstp.reshape(NS, NC, CPW, CHUNK)
    dst64 = dstp.reshape(NW, 2 * CPW, GC)

    xw = pl.pallas_call(
        _mm_body,
        grid=(10,),
        in_specs=[
            pl.BlockSpec((1000, D), lambda i: (i, 0)),
            pl.BlockSpec((D, D), lambda i: (0, 0)),
        ],
        out_specs=pl.BlockSpec((1000, D), lambda i: (i, 0)),
        out_shape=jax.ShapeDtypeStruct((N, D), jnp.float32),
    )(x, W)

    parts, _y = pl.kernel(
        _sc_body,
        out_type=(
            jax.ShapeDtypeStruct((NC, NP, D), jnp.float32),
            jax.ShapeDtypeStruct((NC, N, D), jnp.float32),
        ),
        mesh=plsc.VectorSubcoreMesh(core_axis_name="c", subcore_axis_name="s"),
        scratch_types=[
            pltpu.VMEM((CPW, CHUNK), jnp.int32),
            pltpu.VMEM((2 * CPW, GC), jnp.int32),
            pltpu.VMEM((GC, D), jnp.float32),
            pltpu.VMEM((GC, D), jnp.float32),
            pltpu.VMEM((DRPT,), jnp.float32),
            pltpu.VMEM((DRPT,), jnp.float32),
            pltpu.VMEM((CHUNK,), jnp.float32),
            pltpu.SemaphoreType.DMA,
            pltpu.SemaphoreType.DMA,
            pltpu.VMEM_SHARED((DEGN,), jnp.float32),
            pltpu.VMEM_SHARED((NP, D), jnp.float32),
        ],
        compiler_params=pltpu.CompilerParams(needs_layout_passes=False),
    )(xw, src3, dst4, dst64)

    out = pl.pallas_call(
        _fin_body,
        grid=(10,),
        in_specs=[
            pl.BlockSpec((NC, 1000, D), lambda i: (0, i, 0)),
            pl.BlockSpec((1, D), lambda i: (0, 0)),
        ],
        out_specs=pl.BlockSpec((1000, D), lambda i: (i, 0)),
        out_shape=jax.ShapeDtypeStruct((N, D), jnp.float32),
    )(parts, b.reshape(1, D))
    return out
